# baseline (speedup 1.0000x reference)
import functools

import jax
import jax.numpy as jnp
from jax.experimental import pallas as pl


def _body(emb_ref, anchor_ref, out_ref):
    out_ref[...] = emb_ref[...]

    @pl.when(pl.program_id(1) == 0)
    def _():
        out_ref[0, 0, :] = anchor_ref[0, :]


@functools.partial(jax.jit, static_argnames=("rows_per_block",))
def _run(token_embeddings, style_anchor, rows_per_block=512):
    B, S, D = token_embeddings.shape
    grid = (B, S // rows_per_block)
    return pl.pallas_call(
        _body,
        grid=grid,
        in_specs=[
            pl.BlockSpec((1, rows_per_block, D), lambda b, j: (b, j, 0)),
            pl.BlockSpec((1, D), lambda b, j: (0, 0)),
        ],
        out_specs=pl.BlockSpec((1, rows_per_block, D), lambda b, j: (b, j, 0)),
        out_shape=jax.ShapeDtypeStruct((B, S, D), token_embeddings.dtype),
    )(token_embeddings, style_anchor)


def kernel(token_embeddings, style_anchor):
    return _run(token_embeddings, style_anchor)
